# baseline (speedup 1.0000x reference)
import jax
import jax.numpy as jnp
from jax import lax
from jax.experimental import pallas as pl
from jax.experimental.pallas import tpu as pltpu
from jax.experimental.pallas import tpu_sc as plsc

VOCAB = 49408
D = 512
SEQ = 77
SEQ_PAD = 80
N_CLS = 345
N_STYLE = 8
NC, NS = 2, 16
NW = NC * NS


def _body(tokens_hbm, table_hbm, style_hbm, out_hbm, tok_row, buf, styles_v,
          gsem, wsem, ssem):
    wid = lax.axis_index("s") * NC + lax.axis_index("c")
    c0 = wid * N_CLS // NW
    c1 = (wid + 1) * N_CLS // NW
    pltpu.sync_copy(style_hbm, styles_v)

    def per_class(i, _):
        c = c0 + i
        pltpu.sync_copy(tokens_hbm.at[c], tok_row)
        pltpu.async_copy(table_hbm.at[tok_row], buf, gsem).wait()
        wdescs = [
            pltpu.async_copy(buf.at[pl.ds(0, SEQ)], out_hbm.at[s * N_CLS + c], wsem)
            for s in range(N_STYLE)
        ]
        @pl.when(i > 0)
        def _():
            for s in range(N_STYLE):
                pltpu.make_async_copy(
                    styles_v.at[s], out_hbm.at[s * N_CLS + c - 1, 2], ssem
                ).wait()
        for d in wdescs:
            d.wait()
        for s in range(N_STYLE):
            pltpu.async_copy(styles_v.at[s], out_hbm.at[s * N_CLS + c, 2], ssem)
        return ()

    lax.fori_loop(0, c1 - c0, per_class, ())
    for s in range(N_STYLE):
        pltpu.make_async_copy(styles_v.at[s], out_hbm.at[s * N_CLS + c1 - 1, 2], ssem).wait()


def kernel(tokens, token_table, style_embedding):
    tokens_pad = jnp.pad(tokens, ((0, 0), (0, SEQ_PAD - SEQ)))
    styles = style_embedding.reshape(N_STYLE, D)
    k = pl.kernel(
        _body,
        out_type=jax.ShapeDtypeStruct((N_STYLE * N_CLS, SEQ, D), jnp.float32),
        mesh=plsc.VectorSubcoreMesh(
            core_axis_name="c", subcore_axis_name="s", num_cores=NC, num_subcores=NS
        ),
        scratch_types=[
            pltpu.VMEM((SEQ_PAD,), jnp.int32),
            pltpu.VMEM((SEQ_PAD, D), jnp.float32),
            pltpu.VMEM((N_STYLE, D), jnp.float32),
            pltpu.SemaphoreType.DMA,
            pltpu.SemaphoreType.DMA,
            pltpu.SemaphoreType.DMA,
        ],
        compiler_params=pltpu.CompilerParams(use_tc_tiling_on_sc=False),
    )
    return k(tokens_pad, token_table, styles)


# speedup vs baseline: 3.8017x; 3.8017x over previous
import jax
import jax.numpy as jnp
from jax import lax
from jax.experimental import pallas as pl
from jax.experimental.pallas import tpu as pltpu
from jax.experimental.pallas import tpu_sc as plsc

VOCAB = 49408
D = 512
SEQ = 77
SEQ_PAD = 80
HEAD = 8
TAIL = SEQ - HEAD
TAIL_PAD = SEQ_PAD - HEAD
MID = 64
LAST = TAIL - MID
N_CLS = 345
N_STYLE = 8
NC, NS = 2, 16
NW = NC * NS
LANES = 16


def _body(tokens_hbm, table_hbm, style_hbm, out_hbm, tok8, tok69, ghead, bufb,
          heads, gsem, wsem):
    wid = lax.axis_index("s") * NC + lax.axis_index("c")
    c0 = wid * N_CLS // NW
    c1 = (wid + 1) * N_CLS // NW
    for s in range(N_STYLE):
        pltpu.sync_copy(style_hbm.at[pl.ds(s * D, D)], heads.at[s, 2])

    def per_class(i, _):
        c = c0 + i
        pltpu.sync_copy(tokens_hbm.at[pl.ds(c * SEQ_PAD, HEAD)], tok8)
        pltpu.sync_copy(tokens_hbm.at[pl.ds(c * SEQ_PAD + HEAD, TAIL_PAD)], tok69)
        gh = pltpu.async_copy(table_hbm.at[tok8], ghead, gsem)
        gb = pltpu.async_copy(table_hbm.at[tok69], bufb, gsem)
        gh.wait()

        def fill(j, _):
            for r in (0, 1, 3, 4, 5, 6, 7):
                v = ghead[r, pl.ds(j * LANES, LANES)]
                for s in range(N_STYLE):
                    heads[s, r, pl.ds(j * LANES, LANES)] = v
            return ()

        lax.fori_loop(0, D // LANES, fill, ())
        gb.wait()
        descs = []
        for s in range(N_STYLE):
            row = s * N_CLS + c
            descs.append(
                pltpu.async_copy(heads.at[s], out_hbm.at[row, pl.ds(0, HEAD)], wsem)
            )
            descs.append(
                pltpu.async_copy(
                    bufb.at[pl.ds(0, MID)], out_hbm.at[row, pl.ds(HEAD, MID)], wsem
                )
            )
            descs.append(
                pltpu.async_copy(
                    bufb.at[pl.ds(MID, LAST)],
                    out_hbm.at[row, pl.ds(HEAD + MID, LAST)],
                    wsem,
                )
            )
        for d in descs:
            d.wait()
        return ()

    lax.fori_loop(0, c1 - c0, per_class, ())


def kernel(tokens, token_table, style_embedding):
    tokens_flat = jnp.pad(tokens, ((0, 0), (0, SEQ_PAD - SEQ))).reshape(-1)
    styles_flat = style_embedding.reshape(-1)
    k = pl.kernel(
        _body,
        out_type=jax.ShapeDtypeStruct((N_STYLE * N_CLS, SEQ, D), jnp.float32),
        mesh=plsc.VectorSubcoreMesh(
            core_axis_name="c", subcore_axis_name="s", num_cores=NC, num_subcores=NS
        ),
        scratch_types=[
            pltpu.VMEM((HEAD,), jnp.int32),
            pltpu.VMEM((TAIL_PAD,), jnp.int32),
            pltpu.VMEM((HEAD, D), jnp.float32),
            pltpu.VMEM((TAIL_PAD, D), jnp.float32),
            pltpu.VMEM((N_STYLE, HEAD, D), jnp.float32),
            pltpu.SemaphoreType.DMA,
            pltpu.SemaphoreType.DMA,
        ],
    )
    return k(tokens_flat, token_table, styles_flat)


# speedup vs baseline: 6.8666x; 1.8062x over previous
import jax
import jax.numpy as jnp
from jax import lax
from jax.experimental import pallas as pl
from jax.experimental.pallas import tpu as pltpu
from jax.experimental.pallas import tpu_sc as plsc

VOCAB = 49408
D = 512
SEQ = 77
SEQ_PAD = 80
STYLE_POS = 2
N_CLS = 345
N_STYLE = 8
NC, NS = 2, 16
NW = NC * NS


def _gather_body(tokens_hbm, table_hbm, comp_hbm, tok_row, buf, gsem):
    wid = lax.axis_index("s") * NC + lax.axis_index("c")
    c0 = wid * N_CLS // NW
    c1 = (wid + 1) * N_CLS // NW

    def per_class(i, _):
        c = c0 + i
        pltpu.sync_copy(tokens_hbm.at[pl.ds(c * SEQ_PAD, SEQ_PAD)], tok_row)
        pltpu.async_copy(table_hbm.at[tok_row], buf, gsem).wait()
        pltpu.sync_copy(buf, comp_hbm.at[c])
        return ()

    lax.fori_loop(0, c1 - c0, per_class, ())


def _broadcast_body(comp_ref, style_ref, out_ref):
    r = pl.program_id(0)
    for j in range(8):

        @pl.when(r % 8 == j)
        def _(j=j):
            col = comp_ref[:, j, :]
            for s in range(N_STYLE):
                out_ref[0, pl.ds(s * N_CLS, N_CLS), :] = col

    @pl.when(r == STYLE_POS)
    def _():
        for s in range(N_STYLE):
            out_ref[0, pl.ds(s * N_CLS, N_CLS), :] = jnp.broadcast_to(
                style_ref[s][None, :], (N_CLS, D)
            )


def kernel(tokens, token_table, style_embedding):
    tokens_flat = jnp.pad(tokens, ((0, 0), (0, SEQ_PAD - SEQ))).reshape(-1)
    styles = style_embedding.reshape(N_STYLE, D)

    gather = pl.kernel(
        _gather_body,
        out_type=jax.ShapeDtypeStruct((N_CLS, SEQ_PAD, D), jnp.float32),
        mesh=plsc.VectorSubcoreMesh(
            core_axis_name="c", subcore_axis_name="s", num_cores=NC, num_subcores=NS
        ),
        scratch_types=[
            pltpu.VMEM((SEQ_PAD,), jnp.int32),
            pltpu.VMEM((SEQ_PAD, D), jnp.float32),
            pltpu.SemaphoreType.DMA,
        ],
    )
    compact = gather(tokens_flat, token_table)

    out_t = pl.pallas_call(
        _broadcast_body,
        grid=(SEQ,),
        in_specs=[
            pl.BlockSpec((N_CLS, 8, D), lambda r: (0, r // 8, 0)),
            pl.BlockSpec((N_STYLE, D), lambda r: (0, 0)),
        ],
        out_specs=pl.BlockSpec((1, N_STYLE * N_CLS, D), lambda r: (r, 0, 0)),
        out_shape=jax.ShapeDtypeStruct((SEQ, N_STYLE * N_CLS, D), jnp.float32),
    )(compact, styles)
    return jnp.transpose(out_t, (1, 0, 2))
